# initial kernel scaffold (unmeasured)
import functools

import jax
import jax.numpy as jnp
from jax import lax
from jax.experimental import pallas as pl
from jax.experimental.pallas import tpu as pltpu

N_DEV = 16
B, Sq, Skv, Hq, Dh = 2, 512, 8192, 8, 64
S_SH = Skv // N_DEV
BH = B * Hq


def _ag_body(kt_ref, vt_ref, kf_ref, vf_ref, comm_ref, send_sems, recv_sems,
             copy_sems):
    my = lax.axis_index("i")
    left = (my - 1) % N_DEV
    right = (my + 1) % N_DEV

    barrier_sem = pltpu.get_barrier_semaphore()
    for nbr in (left, right):
        pl.semaphore_signal(
            barrier_sem, inc=1,
            device_id=(nbr,), device_id_type=pl.DeviceIdType.MESH,
        )
    pl.semaphore_wait(barrier_sem, 2)

    comm_ref[0, 0] = kt_ref[...]
    comm_ref[0, 1] = vt_ref[...]
    ck = pltpu.make_async_copy(
        kt_ref, kf_ref.at[:, pl.ds(my * S_SH, S_SH), :], copy_sems.at[0])
    cv = pltpu.make_async_copy(
        vt_ref, vf_ref.at[:, pl.ds(my * S_SH, S_SH), :], copy_sems.at[1])
    ck.start()
    cv.start()
    ck.wait()
    cv.wait()

    for h in range(N_DEV - 1):
        s_slot = h % 2
        r_slot = (h + 1) % 2
        rdma = pltpu.make_async_remote_copy(
            src_ref=comm_ref.at[s_slot],
            dst_ref=comm_ref.at[r_slot],
            send_sem=send_sems.at[s_slot],
            recv_sem=recv_sems.at[r_slot],
            device_id=(right,),
            device_id_type=pl.DeviceIdType.MESH,
        )
        rdma.start()
        rdma.wait()

        origin = (my - h - 1) % N_DEV
        ck = pltpu.make_async_copy(
            comm_ref.at[r_slot, 0],
            kf_ref.at[:, pl.ds(origin * S_SH, S_SH), :], copy_sems.at[0])
        cv = pltpu.make_async_copy(
            comm_ref.at[r_slot, 1],
            vf_ref.at[:, pl.ds(origin * S_SH, S_SH), :], copy_sems.at[1])
        ck.start()
        cv.start()
        ck.wait()
        cv.wait()


def _ag_kv(kt, vt):
    return pl.pallas_call(
        _ag_body,
        out_shape=(
            jax.ShapeDtypeStruct((BH, Skv, Dh), jnp.bfloat16),
            jax.ShapeDtypeStruct((BH, Skv, Dh), jnp.bfloat16),
        ),
        in_specs=[
            pl.BlockSpec(memory_space=pltpu.VMEM),
            pl.BlockSpec(memory_space=pltpu.VMEM),
        ],
        out_specs=(
            pl.BlockSpec(memory_space=pltpu.ANY),
            pl.BlockSpec(memory_space=pltpu.ANY),
        ),
        scratch_shapes=[
            pltpu.VMEM((2, 2, BH, S_SH, Dh), jnp.bfloat16),
            pltpu.SemaphoreType.DMA((2,)),
            pltpu.SemaphoreType.DMA((2,)),
            pltpu.SemaphoreType.DMA((2,)),
        ],
        compiler_params=pltpu.CompilerParams(collective_id=0),
    )(kt, vt)


def _attn_body(x_ref, wq_ref, k_ref, v_ref, wo_ref, out_ref):
    h = pl.program_id(0) % Hq

    @pl.when(h == 0)
    def _():
        out_ref[...] = jnp.zeros_like(out_ref)

    q = lax.dot_general(
        x_ref[...], wq_ref[...],
        (((1,), (0,)), ((), ())),
        preferred_element_type=jnp.float32,
    ).astype(jnp.bfloat16)

    s = lax.dot_general(
        q, k_ref[0],
        (((1,), (1,)), ((), ())),
        preferred_element_type=jnp.float32,
    ) * 0.125

    qb = lax.broadcasted_iota(jnp.int32, (Sq, Skv), 0) // 64
    kb = lax.broadcasted_iota(jnp.int32, (Sq, Skv), 1) // 64
    mask = (qb == kb) | (kb == 0) | ((qb + kb) % 3 == 0)
    s = jnp.where(mask, s, -1e9)

    m = jnp.max(s, axis=-1, keepdims=True)
    w = jnp.exp(s - m)
    l = jnp.sum(w, axis=-1, keepdims=True)
    p = (w * (1.0 / l)).astype(jnp.bfloat16)

    ctx = lax.dot_general(
        p, v_ref[0],
        (((1,), (0,)), ((), ())),
        preferred_element_type=jnp.float32,
    ).astype(jnp.bfloat16)

    out_ref[...] += lax.dot_general(
        ctx, wo_ref[...],
        (((1,), (0,)), ((), ())),
        preferred_element_type=jnp.float32,
    )


def _attn(x2, wq, kf, vf, wo):
    dmodel = x2.shape[1]
    return pl.pallas_call(
        _attn_body,
        grid=(BH,),
        out_shape=jax.ShapeDtypeStruct((B * Sq, dmodel), jnp.float32),
        in_specs=[
            pl.BlockSpec((Sq, dmodel), lambda i: (i // Hq, 0)),
            pl.BlockSpec((dmodel, Dh), lambda i: (0, i % Hq)),
            pl.BlockSpec((1, Skv, Dh), lambda i: (i, 0, 0)),
            pl.BlockSpec((1, Skv, Dh), lambda i: (i, 0, 0)),
            pl.BlockSpec((Dh, dmodel), lambda i: (i % Hq, 0)),
        ],
        out_specs=pl.BlockSpec((Sq, dmodel), lambda i: (i // Hq, 0)),
    )(x2, wq, kf, vf, wo)


def kernel(x, Wq, K_ext, V_ext, Wo):
    dmodel = x.shape[-1]
    x2 = x.reshape(B * Sq, dmodel).astype(jnp.bfloat16)
    wq = Wq.astype(jnp.bfloat16)
    wo = Wo.astype(jnp.bfloat16)
    kt = jnp.transpose(K_ext.astype(jnp.bfloat16), (0, 2, 1, 3)).reshape(
        BH, S_SH, Dh)
    vt = jnp.transpose(V_ext.astype(jnp.bfloat16), (0, 2, 1, 3)).reshape(
        BH, S_SH, Dh)

    kf, vf = _ag_kv(kt, vt)
    out = _attn(x2, wq, kf, vf, wo)
    return out.reshape(B, Sq, dmodel)


# baseline (device time: 998332 ns/iter reference)
import functools

import jax
import jax.numpy as jnp
from jax import lax
from jax.experimental import pallas as pl
from jax.experimental.pallas import tpu as pltpu

N_DEV = 16
B, Sq, Skv, Hq, Dh = 2, 512, 8192, 8, 64
S_SH = Skv // N_DEV
BH = B * Hq


def _ag_body(kt_ref, vt_ref, kf_ref, vf_ref, comm_ref, send_sems, recv_sems,
             copy_sems):
    my = lax.axis_index("i")
    left = (my - 1) % N_DEV
    right = (my + 1) % N_DEV

    barrier_sem = pltpu.get_barrier_semaphore()
    for nbr in (left, right):
        pl.semaphore_signal(
            barrier_sem, inc=1,
            device_id=(nbr,), device_id_type=pl.DeviceIdType.MESH,
        )
    pl.semaphore_wait(barrier_sem, 2)

    comm_ref[0, 0] = kt_ref[...]
    comm_ref[0, 1] = vt_ref[...]
    ck = pltpu.make_async_copy(
        kt_ref, kf_ref.at[:, pl.ds(my * S_SH, S_SH), :], copy_sems.at[0])
    cv = pltpu.make_async_copy(
        vt_ref, vf_ref.at[:, pl.ds(my * S_SH, S_SH), :], copy_sems.at[1])
    ck.start()
    cv.start()
    ck.wait()
    cv.wait()

    for h in range(N_DEV - 1):
        s_slot = h % 2
        r_slot = (h + 1) % 2
        rdma = pltpu.make_async_remote_copy(
            src_ref=comm_ref.at[s_slot],
            dst_ref=comm_ref.at[r_slot],
            send_sem=send_sems.at[s_slot],
            recv_sem=recv_sems.at[r_slot],
            device_id=(right,),
            device_id_type=pl.DeviceIdType.MESH,
        )
        rdma.start()
        rdma.wait()

        origin = (my - h - 1) % N_DEV
        ck = pltpu.make_async_copy(
            comm_ref.at[r_slot, 0],
            kf_ref.at[:, pl.ds(origin * S_SH, S_SH), :], copy_sems.at[0])
        cv = pltpu.make_async_copy(
            comm_ref.at[r_slot, 1],
            vf_ref.at[:, pl.ds(origin * S_SH, S_SH), :], copy_sems.at[1])
        ck.start()
        cv.start()
        ck.wait()
        cv.wait()


def _ag_kv(kt, vt):
    return pl.pallas_call(
        _ag_body,
        out_shape=(
            jax.ShapeDtypeStruct((BH, Skv, Dh), jnp.bfloat16),
            jax.ShapeDtypeStruct((BH, Skv, Dh), jnp.bfloat16),
        ),
        in_specs=[
            pl.BlockSpec(memory_space=pltpu.VMEM),
            pl.BlockSpec(memory_space=pltpu.VMEM),
        ],
        out_specs=(
            pl.BlockSpec(memory_space=pl.ANY),
            pl.BlockSpec(memory_space=pl.ANY),
        ),
        scratch_shapes=[
            pltpu.VMEM((2, 2, BH, S_SH, Dh), jnp.bfloat16),
            pltpu.SemaphoreType.DMA((2,)),
            pltpu.SemaphoreType.DMA((2,)),
            pltpu.SemaphoreType.DMA((2,)),
        ],
        compiler_params=pltpu.CompilerParams(collective_id=0),
    )(kt, vt)


def _attn_body(x_ref, wq_ref, k_ref, v_ref, wo_ref, out_ref):
    h = pl.program_id(0) % Hq

    @pl.when(h == 0)
    def _():
        out_ref[...] = jnp.zeros_like(out_ref)

    q = lax.dot_general(
        x_ref[...], wq_ref[0],
        (((1,), (0,)), ((), ())),
        preferred_element_type=jnp.float32,
    ).astype(jnp.bfloat16)

    s = lax.dot_general(
        q, k_ref[0],
        (((1,), (1,)), ((), ())),
        preferred_element_type=jnp.float32,
    ) * 0.125

    qb = lax.broadcasted_iota(jnp.int32, (Sq, Skv), 0) // 64
    kb = lax.broadcasted_iota(jnp.int32, (Sq, Skv), 1) // 64
    mask = (qb == kb) | (kb == 0) | ((qb + kb) % 3 == 0)
    s = jnp.where(mask, s, -1e9)

    m = jnp.max(s, axis=-1, keepdims=True)
    w = jnp.exp(s - m)
    l = jnp.sum(w, axis=-1, keepdims=True)
    p = (w * (1.0 / l)).astype(jnp.bfloat16)

    ctx = lax.dot_general(
        p, v_ref[0],
        (((1,), (0,)), ((), ())),
        preferred_element_type=jnp.float32,
    ).astype(jnp.bfloat16)

    out_ref[...] += lax.dot_general(
        ctx, wo_ref[0],
        (((1,), (0,)), ((), ())),
        preferred_element_type=jnp.float32,
    )


def _attn(x2, wq3, kf, vf, wo3):
    dmodel = x2.shape[1]
    return pl.pallas_call(
        _attn_body,
        grid=(BH,),
        out_shape=jax.ShapeDtypeStruct((B * Sq, dmodel), jnp.float32),
        in_specs=[
            pl.BlockSpec((Sq, dmodel), lambda i: (i // Hq, 0)),
            pl.BlockSpec((1, dmodel, Dh), lambda i: (i % Hq, 0, 0)),
            pl.BlockSpec((1, Skv, Dh), lambda i: (i, 0, 0)),
            pl.BlockSpec((1, Skv, Dh), lambda i: (i, 0, 0)),
            pl.BlockSpec((1, Dh, dmodel), lambda i: (i % Hq, 0, 0)),
        ],
        out_specs=pl.BlockSpec((Sq, dmodel), lambda i: (i // Hq, 0)),
    )(x2, wq3, kf, vf, wo3)


def kernel(x, Wq, K_ext, V_ext, Wo):
    dmodel = x.shape[-1]
    x2 = x.reshape(B * Sq, dmodel).astype(jnp.bfloat16)
    wq3 = jnp.transpose(
        Wq.astype(jnp.bfloat16).reshape(dmodel, Hq, Dh), (1, 0, 2))
    wo3 = Wo.astype(jnp.bfloat16).reshape(Hq, Dh, dmodel)
    kt = jnp.transpose(K_ext.astype(jnp.bfloat16), (0, 2, 1, 3)).reshape(
        BH, S_SH, Dh)
    vt = jnp.transpose(V_ext.astype(jnp.bfloat16), (0, 2, 1, 3)).reshape(
        BH, S_SH, Dh)

    kf, vf = _ag_kv(kt, vt)
    out = _attn(x2, wq3, kf, vf, wo3)
    return out.reshape(B, Sq, dmodel)


# device time: 246289 ns/iter; 4.0535x vs baseline; 4.0535x over previous
import jax
import jax.numpy as jnp
from jax import lax
from jax.experimental import pallas as pl
from jax.experimental.pallas import tpu as pltpu

N_DEV = 16
B, Sq, Skv, Hq, Dh = 2, 512, 8192, 8, 64
S_SH = Skv // N_DEV
BH = B * Hq
_HALF = N_DEV // 2


def _fused_body(x_ref, wq_ref, kt_ref, vt_ref, wo_ref, bias_ref, out_ref,
                commR, commL, q_ref, acc_ref, l_ref,
                sendR, recvR, sendL, recvL):
    my = lax.axis_index("i")
    left = (my - 1) % N_DEV
    right = (my + 1) % N_DEV

    barrier_sem = pltpu.get_barrier_semaphore()
    for nbr in (left, right):
        pl.semaphore_signal(
            barrier_sem, inc=1,
            device_id=(nbr,), device_id_type=pl.DeviceIdType.MESH,
        )
    pl.semaphore_wait(barrier_sem, 2)

    commR[0, 0] = kt_ref[...]
    commR[0, 1] = vt_ref[...]
    commL[0, 0] = kt_ref[...]
    commL[0, 1] = vt_ref[...]

    for b in range(B):
        xb = x_ref[b * Sq:(b + 1) * Sq, :]
        for h in range(Hq):
            q_ref[b * Hq + h] = lax.dot_general(
                xb, wq_ref[h], (((1,), (0,)), ((), ())),
                preferred_element_type=jnp.float32,
            ).astype(jnp.bfloat16)
    acc_ref[...] = jnp.zeros_like(acc_ref)
    l_ref[...] = jnp.zeros_like(l_ref)

    def flash_step(k_src, v_src, origin):
        bias = bias_ref[origin].astype(jnp.float32)
        for i in range(BH):
            s = lax.dot_general(
                q_ref[i], k_src[i], (((1,), (0,)), ((), ())),
                preferred_element_type=jnp.float32,
            )
            w = jnp.exp(s * 0.125 + bias)
            l_ref[i] = l_ref[i] + jnp.sum(w, axis=1, keepdims=True)
            acc_ref[i] = acc_ref[i] + lax.dot_general(
                w.astype(jnp.bfloat16), v_src[i], (((1,), (1,)), ((), ())),
                preferred_element_type=jnp.float32,
            )

    def rdma(comm, s_slot, r_slot, send_sems, recv_sems, dev):
        return pltpu.make_async_remote_copy(
            src_ref=comm.at[s_slot],
            dst_ref=comm.at[r_slot],
            send_sem=send_sems.at[s_slot],
            recv_sem=recv_sems.at[r_slot],
            device_id=(dev,),
            device_id_type=pl.DeviceIdType.MESH,
        )

    rR = rdma(commR, 0, 1, sendR, recvR, right)
    rL = rdma(commL, 0, 1, sendL, recvL, left)
    rR.start()
    rL.start()
    flash_step(kt_ref, vt_ref, my)
    rR.wait()
    rL.wait()

    for j in range(1, _HALF):
        s_slot = j % 2
        r_slot = (j + 1) % 2
        rR = rdma(commR, s_slot, r_slot, sendR, recvR, right)
        rR.start()
        if j < _HALF - 1:
            rL = rdma(commL, s_slot, r_slot, sendL, recvL, left)
            rL.start()
        flash_step(commR.at[s_slot, 0], commR.at[s_slot, 1], (my - j) % N_DEV)
        flash_step(commL.at[s_slot, 0], commL.at[s_slot, 1], (my + j) % N_DEV)
        rR.wait()
        if j < _HALF - 1:
            rL.wait()

    flash_step(commR.at[0, 0], commR.at[0, 1], (my - _HALF) % N_DEV)

    for b in range(B):
        o = jnp.zeros((Sq, out_ref.shape[1]), jnp.float32)
        for h in range(Hq):
            i = b * Hq + h
            ctx = (acc_ref[i] * (1.0 / l_ref[i])).astype(jnp.bfloat16)
            o = o + lax.dot_general(
                ctx, wo_ref[h], (((1,), (0,)), ((), ())),
                preferred_element_type=jnp.float32,
            )
        out_ref[b * Sq:(b + 1) * Sq, :] = o


def kernel(x, Wq, K_ext, V_ext, Wo):
    dmodel = x.shape[-1]
    x2 = x.reshape(B * Sq, dmodel).astype(jnp.bfloat16)
    wq3 = jnp.transpose(
        Wq.astype(jnp.bfloat16).reshape(dmodel, Hq, Dh), (1, 0, 2))
    wo3 = Wo.astype(jnp.bfloat16).reshape(Hq, Dh, dmodel)
    kt = jnp.transpose(K_ext.astype(jnp.bfloat16), (0, 2, 3, 1)).reshape(
        BH, Dh, S_SH)
    vt = jnp.transpose(V_ext.astype(jnp.bfloat16), (0, 2, 3, 1)).reshape(
        BH, Dh, S_SH)

    qb = (jnp.arange(Sq) // 64)[None, :, None]
    kb = ((jnp.arange(N_DEV) * S_SH)[:, None, None]
          + jnp.arange(S_SH)[None, None, :]) // 64
    keep = (qb == kb) | (kb == 0) | ((qb + kb) % 3 == 0)
    bias = jnp.where(keep, 0.0, -1e9).astype(jnp.bfloat16)

    out = pl.pallas_call(
        _fused_body,
        out_shape=jax.ShapeDtypeStruct((B * Sq, dmodel), jnp.float32),
        in_specs=[pl.BlockSpec(memory_space=pltpu.VMEM)] * 6,
        out_specs=pl.BlockSpec(memory_space=pltpu.VMEM),
        scratch_shapes=[
            pltpu.VMEM((2, 2, BH, Dh, S_SH), jnp.bfloat16),
            pltpu.VMEM((2, 2, BH, Dh, S_SH), jnp.bfloat16),
            pltpu.VMEM((BH, Sq, Dh), jnp.bfloat16),
            pltpu.VMEM((BH, Sq, Dh), jnp.float32),
            pltpu.VMEM((BH, Sq, 1), jnp.float32),
            pltpu.SemaphoreType.DMA((2,)),
            pltpu.SemaphoreType.DMA((2,)),
            pltpu.SemaphoreType.DMA((2,)),
            pltpu.SemaphoreType.DMA((2,)),
        ],
        compiler_params=pltpu.CompilerParams(collective_id=0),
    )(x2, wq3, kt, vt, wo3, bias)
    return out.reshape(B, Sq, dmodel)


# device time: 223817 ns/iter; 4.4605x vs baseline; 1.1004x over previous
import jax
import jax.numpy as jnp
from jax import lax
from jax.experimental import pallas as pl
from jax.experimental.pallas import tpu as pltpu

N_DEV = 16
B, Sq, Skv, Hq, Dh = 2, 512, 8192, 8, 64
S_SH = Skv // N_DEV
BH = B * Hq
_HALF = N_DEV // 2
_HS = S_SH // 2


def _fused_body(x_ref, wq_ref, kt_ref, vt_ref, wo_ref, bias_ref, out_ref,
                commR, commL, q_ref, acc_ref, l_ref,
                sendR, recvR, sendL, recvL):
    my = lax.axis_index("i")
    left = (my - 1) % N_DEV
    right = (my + 1) % N_DEV

    barrier_sem = pltpu.get_barrier_semaphore()
    for nbr in (left, right):
        pl.semaphore_signal(
            barrier_sem, inc=1,
            device_id=(nbr,), device_id_type=pl.DeviceIdType.MESH,
        )
    pl.semaphore_wait(barrier_sem, 2)

    commR[0, 0] = kt_ref[...]
    commR[0, 1] = vt_ref[...]
    commL[0, 0] = kt_ref[...]
    commL[0, 1] = vt_ref[...]

    for b in range(B):
        xb = x_ref[b * Sq:(b + 1) * Sq, :]
        for h in range(Hq):
            q_ref[b * Hq + h] = lax.dot_general(
                xb, wq_ref[h], (((1,), (0,)), ((), ())),
                preferred_element_type=jnp.float32,
            ).astype(jnp.bfloat16)
    acc_ref[...] = jnp.zeros_like(acc_ref)
    l_ref[...] = jnp.zeros_like(l_ref)

    def flash_step(k_src, v_src, origin):
        bias = bias_ref[origin].astype(jnp.float32)
        for i in range(BH):
            s = lax.dot_general(
                q_ref[i], k_src[i], (((1,), (0,)), ((), ())),
                preferred_element_type=jnp.float32,
            )
            w = jnp.exp(s * 0.125 + bias)
            l_ref[i] = l_ref[i] + jnp.sum(w, axis=1, keepdims=True)
            acc_ref[i] = acc_ref[i] + lax.dot_general(
                w.astype(jnp.bfloat16), v_src[i], (((1,), (1,)), ((), ())),
                preferred_element_type=jnp.float32,
            )

    def rdma_half(comm, s_slot, r_slot, half, send_sems, recv_sems, dev):
        ds = pl.ds(half * _HS, _HS)
        return pltpu.make_async_remote_copy(
            src_ref=comm.at[s_slot, :, :, :, ds],
            dst_ref=comm.at[r_slot, :, :, :, ds],
            send_sem=send_sems.at[s_slot, half],
            recv_sem=recv_sems.at[r_slot, half],
            device_id=(dev,),
            device_id_type=pl.DeviceIdType.MESH,
        )

    rA = rdma_half(commR, 0, 1, 0, sendR, recvR, right)
    rB = rdma_half(commR, 0, 1, 1, sendR, recvR, right)
    lA = rdma_half(commL, 0, 1, 0, sendL, recvL, left)
    lB = rdma_half(commL, 0, 1, 1, sendL, recvL, left)
    rA.start()
    rB.start()
    lA.start()
    lB.start()
    flash_step(kt_ref, vt_ref, my)

    for j in range(1, _HALF):
        s_slot = j % 2
        r_slot = (j + 1) % 2
        rA.wait()
        rA = rdma_half(commR, s_slot, r_slot, 0, sendR, recvR, right)
        rA.start()
        rB.wait()
        rB = rdma_half(commR, s_slot, r_slot, 1, sendR, recvR, right)
        rB.start()
        lA.wait()
        lB.wait()
        if j < _HALF - 1:
            lA = rdma_half(commL, s_slot, r_slot, 0, sendL, recvL, left)
            lA.start()
            lB = rdma_half(commL, s_slot, r_slot, 1, sendL, recvL, left)
            lB.start()
        flash_step(commR.at[s_slot, 0], commR.at[s_slot, 1], (my - j) % N_DEV)
        flash_step(commL.at[s_slot, 0], commL.at[s_slot, 1], (my + j) % N_DEV)

    rA.wait()
    rB.wait()
    flash_step(commR.at[0, 0], commR.at[0, 1], (my - _HALF) % N_DEV)

    for b in range(B):
        o = jnp.zeros((Sq, out_ref.shape[1]), jnp.float32)
        for h in range(Hq):
            i = b * Hq + h
            ctx = (acc_ref[i] * (1.0 / l_ref[i])).astype(jnp.bfloat16)
            o = o + lax.dot_general(
                ctx, wo_ref[h], (((1,), (0,)), ((), ())),
                preferred_element_type=jnp.float32,
            )
        out_ref[b * Sq:(b + 1) * Sq, :] = o


def kernel(x, Wq, K_ext, V_ext, Wo):
    dmodel = x.shape[-1]
    x2 = x.reshape(B * Sq, dmodel).astype(jnp.bfloat16)
    wq3 = jnp.transpose(
        Wq.astype(jnp.bfloat16).reshape(dmodel, Hq, Dh), (1, 0, 2))
    wo3 = Wo.astype(jnp.bfloat16).reshape(Hq, Dh, dmodel)
    kt = jnp.transpose(K_ext.astype(jnp.bfloat16), (0, 2, 3, 1)).reshape(
        BH, Dh, S_SH)
    vt = jnp.transpose(V_ext.astype(jnp.bfloat16), (0, 2, 3, 1)).reshape(
        BH, Dh, S_SH)

    qb = (jnp.arange(Sq) // 64)[None, :, None]
    kb = ((jnp.arange(N_DEV) * S_SH)[:, None, None]
          + jnp.arange(S_SH)[None, None, :]) // 64
    keep = (qb == kb) | (kb == 0) | ((qb + kb) % 3 == 0)
    bias = jnp.where(keep, 0.0, -1e9).astype(jnp.bfloat16)

    out = pl.pallas_call(
        _fused_body,
        out_shape=jax.ShapeDtypeStruct((B * Sq, dmodel), jnp.float32),
        in_specs=[pl.BlockSpec(memory_space=pltpu.VMEM)] * 6,
        out_specs=pl.BlockSpec(memory_space=pltpu.VMEM),
        scratch_shapes=[
            pltpu.VMEM((2, 2, BH, Dh, S_SH), jnp.bfloat16),
            pltpu.VMEM((2, 2, BH, Dh, S_SH), jnp.bfloat16),
            pltpu.VMEM((BH, Sq, Dh), jnp.bfloat16),
            pltpu.VMEM((BH, Sq, Dh), jnp.float32),
            pltpu.VMEM((BH, Sq, 1), jnp.float32),
            pltpu.SemaphoreType.DMA((2, 2)),
            pltpu.SemaphoreType.DMA((2, 2)),
            pltpu.SemaphoreType.DMA((2, 2)),
            pltpu.SemaphoreType.DMA((2, 2)),
        ],
        compiler_params=pltpu.CompilerParams(collective_id=0),
    )(x2, wq3, kt, vt, wo3, bias)
    return out.reshape(B, Sq, dmodel)
